# Initial kernel scaffold; baseline (speedup 1.0000x reference)
import functools

import jax
import jax.numpy as jnp
from jax import lax
from jax.experimental import pallas as pl
from jax.experimental.pallas import tpu as pltpu
from jax.experimental.pallas import tpu_sc as plsc

N = 10000
E = 320000
D = 128
H = 128
G = 64

NC = 2
NS = 16
NW = NC * NS
CH = 128
KCH = 79
EW = KCH * CH
E_PAD = NW * EW
NP = 10016
TR = NP // NS

R = 1000
GRID = N // R


def _sc_scatter_body(x_hbm, src_hbm, dst_hbm, zero_hbm, out_hbm,
                     acc, src_v, dst_v, rows_v, sem):
    c = lax.axis_index("c")
    s = lax.axis_index("s")
    wid = s * NC + c

    pltpu.sync_copy(zero_hbm.at[pl.ds(s * TR, TR)], acc.at[pl.ds(s * TR, TR)])
    pltpu.sync_copy(src_hbm.at[wid], src_v)
    pltpu.sync_copy(dst_hbm.at[wid], dst_v)
    plsc.subcore_barrier()

    def step(i, carry):
        pltpu.async_copy(x_hbm.at[src_v.at[i]], rows_v, sem).wait()
        pltpu.sync_copy(rows_v, acc.at[dst_v.at[i]], add=True)
        return carry

    lax.fori_loop(0, KCH, step, 0)
    plsc.subcore_barrier()
    pltpu.sync_copy(acc.at[pl.ds(s * TR, TR)],
                    out_hbm.at[c, pl.ds(s * TR, TR)])


@jax.jit
def _sc_scatter(x, srcp, dstp, zero):
    mesh = plsc.VectorSubcoreMesh(core_axis_name="c", subcore_axis_name="s")
    fn = pl.kernel(
        _sc_scatter_body,
        out_type=jax.ShapeDtypeStruct((NC, NP, D), jnp.float32),
        mesh=mesh,
        scratch_types=[
            pltpu.VMEM_SHARED((NP, D), jnp.float32),
            pltpu.VMEM((KCH, CH), jnp.int32),
            pltpu.VMEM((KCH, CH), jnp.int32),
            pltpu.VMEM((CH, D), jnp.float32),
            pltpu.SemaphoreType.DMA,
        ],
    )
    return fn(x, srcp, dstp, zero)


def _tc1_body(agg_a, agg_b, x, w_rel, w_root, b, out):
    agg = agg_a[0] + agg_b[0]
    h = (jnp.dot(agg, w_rel[...], preferred_element_type=jnp.float32)
         + jnp.dot(x[...], w_root[...], preferred_element_type=jnp.float32)
         + b[...])
    out[...] = jnp.maximum(h, 0.0)


@jax.jit
def _tc_layer1(agg, x, w_rel, w_root, b):
    return pl.pallas_call(
        _tc1_body,
        grid=(GRID,),
        in_specs=[
            pl.BlockSpec((1, R, D), lambda i: (0, i, 0)),
            pl.BlockSpec((1, R, D), lambda i: (1, i, 0)),
            pl.BlockSpec((R, D), lambda i: (i, 0)),
            pl.BlockSpec((D, H), lambda i: (0, 0)),
            pl.BlockSpec((D, H), lambda i: (0, 0)),
            pl.BlockSpec((1, H), lambda i: (0, 0)),
        ],
        out_specs=pl.BlockSpec((R, H), lambda i: (i, 0)),
        out_shape=jax.ShapeDtypeStruct((N, H), jnp.float32),
    )(agg, agg, x, w_rel, w_root, b)


def _tc2_body(agg_a, agg_b, h, batch, w_rel, w_root, b, wl, bl, out,
              sums, counts):
    i = pl.program_id(0)

    @pl.when(i == 0)
    def _():
        sums[...] = jnp.zeros((G, H), jnp.float32)
        counts[...] = jnp.zeros((G, H), jnp.float32)

    agg = agg_a[0] + agg_b[0]
    h2 = (jnp.dot(agg, w_rel[...], preferred_element_type=jnp.float32)
          + jnp.dot(h[...], w_root[...], preferred_element_type=jnp.float32)
          + b[...])
    h2 = jnp.maximum(h2, 0.0)

    bvec = batch[0, 0, :]
    oh = (lax.broadcasted_iota(jnp.int32, (G, R), 0)
          == bvec[None, :]).astype(jnp.float32)
    sums[...] += jnp.dot(oh, h2, preferred_element_type=jnp.float32)
    counts[...] += jnp.dot(oh, jnp.ones((R, H), jnp.float32),
                           preferred_element_type=jnp.float32)

    @pl.when(i == GRID - 1)
    def _():
        pooled = sums[...] / jnp.maximum(counts[...], 1.0)
        out[...] = (jnp.dot(pooled, wl[...],
                            preferred_element_type=jnp.float32) + bl[...])


@jax.jit
def _tc_layer2_pool(agg, h, batch3, w_rel, w_root, b, wl, bl):
    return pl.pallas_call(
        _tc2_body,
        grid=(GRID,),
        in_specs=[
            pl.BlockSpec((1, R, D), lambda i: (0, i, 0)),
            pl.BlockSpec((1, R, D), lambda i: (1, i, 0)),
            pl.BlockSpec((R, H), lambda i: (i, 0)),
            pl.BlockSpec((1, 1, R), lambda i: (i, 0, 0)),
            pl.BlockSpec((H, H), lambda i: (0, 0)),
            pl.BlockSpec((H, H), lambda i: (0, 0)),
            pl.BlockSpec((1, H), lambda i: (0, 0)),
            pl.BlockSpec((H, 1), lambda i: (0, 0)),
            pl.BlockSpec((1, 1), lambda i: (0, 0)),
        ],
        out_specs=pl.BlockSpec((G, 1), lambda i: (0, 0)),
        out_shape=jax.ShapeDtypeStruct((G, 1), jnp.float32),
        scratch_shapes=[
            pltpu.VMEM((G, H), jnp.float32),
            pltpu.VMEM((G, H), jnp.float32),
        ],
    )(agg, agg, h, batch3, w_rel, w_root, b, wl, bl)


def kernel(x, edge_index, batch, W1_rel, W1_root, b1, W2_rel, W2_root, b2,
           Wl, bl):
    src = edge_index[0]
    dst = edge_index[1]
    pad = E_PAD - E
    srcp = jnp.concatenate(
        [src, jnp.zeros((pad,), jnp.int32)]).reshape(NW, KCH, CH)
    dstp = jnp.concatenate(
        [dst, jnp.full((pad,), N, jnp.int32)]).reshape(NW, KCH, CH)
    zero = jnp.zeros((NP, D), jnp.float32)

    agg1 = _sc_scatter(x, srcp, dstp, zero)
    h = _tc_layer1(agg1, x, W1_rel, W1_root, b1.reshape(1, H))
    agg2 = _sc_scatter(h, srcp, dstp, zero)
    out = _tc_layer2_pool(agg2, h, batch.reshape(GRID, 1, R),
                          W2_rel, W2_root, b2.reshape(1, H),
                          Wl, bl.reshape(1, 1))
    return out


# speedup vs baseline: 4.6140x; 4.6140x over previous
import functools

import jax
import jax.numpy as jnp
from jax import lax
from jax.experimental import pallas as pl
from jax.experimental.pallas import tpu as pltpu
from jax.experimental.pallas import tpu_sc as plsc

N = 10000
E = 320000
D = 128
H = 128
G = 64

NC = 2
NS = 16
NW = NC * NS
CH = 128
KCH = 79
EW = KCH * CH
E_PAD = NW * EW
NP = 10112
TR = NP // NS

R = 1000
GRID = N // R


def _sc_scatter_body(x_hbm, src_hbm, dst_hbm, zero_hbm, out_hbm,
                     acc, src_v, dst_v, rows_v, sem):
    c = lax.axis_index("c")
    s = lax.axis_index("s")
    wid = s * NC + c

    pltpu.sync_copy(zero_hbm.at[pl.ds(s * TR, TR)], acc.at[pl.ds(s * TR, TR)])
    pltpu.sync_copy(src_hbm.at[wid], src_v)
    pltpu.sync_copy(dst_hbm.at[wid], dst_v)
    plsc.subcore_barrier()

    def step(i, carry):
        pltpu.async_copy(x_hbm.at[src_v.at[i]], rows_v, sem).wait()
        pltpu.sync_copy(rows_v, acc.at[dst_v.at[i]], add=True)
        return carry

    lax.fori_loop(0, KCH, step, 0)
    plsc.subcore_barrier()
    pltpu.sync_copy(acc.at[pl.ds(s * TR, TR)],
                    out_hbm.at[c, pl.ds(s * TR, TR)])


@jax.jit
def _sc_scatter(x, srcp, dstp, zero):
    mesh = plsc.VectorSubcoreMesh(core_axis_name="c", subcore_axis_name="s",
                                  num_cores=NC, num_subcores=NS)
    fn = pl.kernel(
        _sc_scatter_body,
        out_type=jax.ShapeDtypeStruct((NC, NP, D), jnp.float32),
        mesh=mesh,
        scratch_types=[
            pltpu.VMEM_SHARED((NP, D), jnp.float32),
            pltpu.VMEM((KCH, CH), jnp.int32),
            pltpu.VMEM((KCH, CH), jnp.int32),
            pltpu.VMEM((CH, D), jnp.float32),
            pltpu.SemaphoreType.DMA,
        ],
    )
    return fn(x, srcp, dstp, zero)


def _tc1_body(agg_a, agg_b, x, w_rel, w_root, b, out):
    agg = agg_a[0] + agg_b[0]
    h = (jnp.dot(agg, w_rel[...], preferred_element_type=jnp.float32)
         + jnp.dot(x[...], w_root[...], preferred_element_type=jnp.float32)
         + b[...])
    out[...] = jnp.maximum(h, 0.0)


@jax.jit
def _tc_layer1(agg, x, w_rel, w_root, b):
    return pl.pallas_call(
        _tc1_body,
        grid=(GRID,),
        in_specs=[
            pl.BlockSpec((1, R, D), lambda i: (0, i, 0)),
            pl.BlockSpec((1, R, D), lambda i: (1, i, 0)),
            pl.BlockSpec((R, D), lambda i: (i, 0)),
            pl.BlockSpec((D, H), lambda i: (0, 0)),
            pl.BlockSpec((D, H), lambda i: (0, 0)),
            pl.BlockSpec((1, H), lambda i: (0, 0)),
        ],
        out_specs=pl.BlockSpec((R, H), lambda i: (i, 0)),
        out_shape=jax.ShapeDtypeStruct((N, H), jnp.float32),
    )(agg, agg, x, w_rel, w_root, b)


def _tc2_body(agg_a, agg_b, h, batch, w_rel, w_root, b, wl, bl, out,
              sums, counts):
    i = pl.program_id(0)

    @pl.when(i == 0)
    def _():
        sums[...] = jnp.zeros((G, H), jnp.float32)
        counts[...] = jnp.zeros((G, H), jnp.float32)

    agg = agg_a[0] + agg_b[0]
    h2 = (jnp.dot(agg, w_rel[...], preferred_element_type=jnp.float32)
          + jnp.dot(h[...], w_root[...], preferred_element_type=jnp.float32)
          + b[...])
    h2 = jnp.maximum(h2, 0.0)

    bvec = batch[0, 0, :]
    oh = (lax.broadcasted_iota(jnp.int32, (G, R), 0)
          == bvec[None, :]).astype(jnp.float32)
    sums[...] += jnp.dot(oh, h2, preferred_element_type=jnp.float32)
    counts[...] += jnp.dot(oh, jnp.ones((R, H), jnp.float32),
                           preferred_element_type=jnp.float32)

    @pl.when(i == GRID - 1)
    def _():
        pooled = sums[...] / jnp.maximum(counts[...], 1.0)
        out[...] = (jnp.dot(pooled, wl[...],
                            preferred_element_type=jnp.float32) + bl[...])


@jax.jit
def _tc_layer2_pool(agg, h, batch3, w_rel, w_root, b, wl, bl):
    return pl.pallas_call(
        _tc2_body,
        grid=(GRID,),
        in_specs=[
            pl.BlockSpec((1, R, D), lambda i: (0, i, 0)),
            pl.BlockSpec((1, R, D), lambda i: (1, i, 0)),
            pl.BlockSpec((R, H), lambda i: (i, 0)),
            pl.BlockSpec((1, 1, R), lambda i: (i, 0, 0)),
            pl.BlockSpec((H, H), lambda i: (0, 0)),
            pl.BlockSpec((H, H), lambda i: (0, 0)),
            pl.BlockSpec((1, H), lambda i: (0, 0)),
            pl.BlockSpec((H, 1), lambda i: (0, 0)),
            pl.BlockSpec((1, 1), lambda i: (0, 0)),
        ],
        out_specs=pl.BlockSpec((G, 1), lambda i: (0, 0)),
        out_shape=jax.ShapeDtypeStruct((G, 1), jnp.float32),
        scratch_shapes=[
            pltpu.VMEM((G, H), jnp.float32),
            pltpu.VMEM((G, H), jnp.float32),
        ],
    )(agg, agg, h, batch3, w_rel, w_root, b, wl, bl)


def kernel(x, edge_index, batch, W1_rel, W1_root, b1, W2_rel, W2_root, b2,
           Wl, bl):
    src = edge_index[0]
    dst = edge_index[1]
    pad = E_PAD - E
    srcp = jnp.concatenate(
        [src, jnp.zeros((pad,), jnp.int32)]).reshape(NW, KCH, CH)
    dstp = jnp.concatenate(
        [dst, jnp.full((pad,), N, jnp.int32)]).reshape(NW, KCH, CH)
    zero = jnp.zeros((NP, D), jnp.float32)

    agg1 = _sc_scatter(x, srcp, dstp, zero)
    h = _tc_layer1(agg1, x, W1_rel, W1_root, b1.reshape(1, H))
    agg2 = _sc_scatter(h, srcp, dstp, zero)
    out = _tc_layer2_pool(agg2, h, batch.reshape(GRID, 1, R),
                          W2_rel, W2_root, b2.reshape(1, H),
                          Wl, bl.reshape(1, 1))
    return out
